# Initial kernel scaffold; baseline (speedup 1.0000x reference)
import jax
import jax.numpy as jnp
from jax.experimental import pallas as pl


def kernel(x, W0, b0, Wl1, bl1, Wr1, br1, att1, bias1, Wl2, bl2, Wr2, br2, att2, bias2, Wout, bout, edge_index):
    raise NotImplementedError("write your pallas kernel here")


# speedup vs baseline: 1.0917x; 1.0917x over previous
import functools

import jax
import jax.numpy as jnp
from jax.experimental import pallas as pl
from jax.experimental.pallas import tpu as pltpu

N = 10000
IN = 256
HID = 256
OUT = 256
HEADS = 4


def _mm_body(x_ref, w_ref, b_ref, o_ref, *, slope):
    acc = jnp.dot(x_ref[...], w_ref[...], preferred_element_type=jnp.float32)
    acc = acc + b_ref[...]
    if slope is not None:
        acc = jnp.where(acc >= 0, acc, slope * acc)
    o_ref[...] = acc


def _matmul(x, w, b, slope=None, bm=1000, bn=512):
    m, k = x.shape
    n = w.shape[1]
    bn = min(bn, n)
    grid = (m // bm, n // bn)
    return pl.pallas_call(
        functools.partial(_mm_body, slope=slope),
        grid=grid,
        in_specs=[
            pl.BlockSpec((bm, k), lambda i, j: (i, 0)),
            pl.BlockSpec((k, bn), lambda i, j: (0, j)),
            pl.BlockSpec((1, bn), lambda i, j: (0, j)),
        ],
        out_specs=pl.BlockSpec((bm, bn), lambda i, j: (i, j)),
        out_shape=jax.ShapeDtypeStruct((m, n), jnp.float32),
    )(x, w, b.reshape(1, n))


def _edge_stage(xl, xr, src, dst, att, bias):
    n = xl.shape[0]
    xl = xl.reshape(n, HEADS, HID)
    xr = xr.reshape(n, HEADS, HID)
    e = jax.nn.leaky_relu(xl[src] + xr[dst], 0.2)
    alpha = (e * att[None]).sum(-1)
    ex = jnp.exp(alpha)
    esum = jax.ops.segment_sum(ex, dst, num_segments=n) + 1e-16
    num = jax.ops.segment_sum(xl[src] * ex[:, :, None], dst, num_segments=n)
    out = num / esum[:, :, None]
    return out.mean(axis=1) + bias


def kernel(x, W0, b0, Wl1, bl1, Wr1, br1, att1, bias1, Wl2, bl2, Wr2, br2,
           att2, bias2, Wout, bout, edge_index):
    n = x.shape[0]
    loop = jnp.arange(n, dtype=edge_index.dtype)
    src = jnp.concatenate([edge_index[0], loop])
    dst = jnp.concatenate([edge_index[1], loop])

    h = _matmul(x, W0, b0, slope=0.01)

    for (Wl, bl, Wr, br, att, bias) in (
            (Wl1, bl1, Wr1, br1, att1, bias1),
            (Wl2, bl2, Wr2, br2, att2, bias2)):
        xl = _matmul(h, Wl, bl)
        xr = _matmul(h, Wr, br)
        g = _edge_stage(xl, xr, src, dst, att, bias)
        h = jax.nn.leaky_relu(g, 0.01)

    out = _matmul(h, Wout, bout)
    return out.mean(axis=0)


# speedup vs baseline: 6.8541x; 6.2782x over previous
import functools

import jax
import jax.numpy as jnp
from jax import lax
from jax.experimental import pallas as pl
from jax.experimental.pallas import tpu as pltpu
from jax.experimental.pallas import tpu_sc as plsc

N = 10000
IN = 256
HID = 256
OUT = 256
HEADS = 4
HALF = HID // 2
FH = HEADS * HALF

NC = 2
NS = 16
NW = NC * NS
L = 16
CH = 16
E2 = 170000
EPAD = 171008
EW1 = EPAD // NW
EW2 = EPAD // NS
NP = 10240
NROW = NP // NS
RW = 128

_DN = lax.GatherDimensionNumbers(
    offset_dims=(), collapsed_slice_dims=(0,), start_index_map=(0,))


def _shuf(v, idx):
    return lax.gather(v, idx.reshape(L, 1), _DN, slice_sizes=(1,),
                      mode=lax.GatherScatterMode.PROMISE_IN_BOUNDS)


def _lane_sum(v):
    for sh in (8, 4, 2, 1):
        v = v + _shuf(v, lax.iota(jnp.int32, L) ^ sh)
    return v


def _bcast_lane(v, i):
    return _shuf(v, jnp.full((L,), i, jnp.int32))


def _mm_body(x_ref, w_ref, b_ref, o_ref, *, slope, in_bias, in_slope):
    x = x_ref[...]
    if in_bias is not None:
        x = x + in_bias[...]
        x = jnp.where(x >= 0, x, in_slope * x)
    acc = jnp.dot(x, w_ref[...], preferred_element_type=jnp.float32)
    acc = acc + b_ref[...]
    if slope is not None:
        acc = jnp.where(acc >= 0, acc, slope * acc)
    o_ref[...] = acc


def _matmul(x, w, b, slope=None, in_bias=None, in_slope=0.01,
            bm=1000, bn=512):
    m, k = x.shape
    n = w.shape[1]
    bn = min(bn, n)
    grid = (m // bm, n // bn)
    in_specs = [
        pl.BlockSpec((bm, k), lambda i, j: (i, 0)),
        pl.BlockSpec((k, bn), lambda i, j: (0, j)),
        pl.BlockSpec((1, bn), lambda i, j: (0, j)),
    ]
    args = [x, w, b.reshape(1, n)]
    if in_bias is not None:
        in_specs.append(pl.BlockSpec((1, k), lambda i, j: (0, 0)))
        args.append(in_bias.reshape(1, k))

    def body(x_ref, w_ref, b_ref, *rest):
        if in_bias is not None:
            ib_ref, o_ref = rest
        else:
            ib_ref, o_ref = None, rest[0]
        _mm_body(x_ref, w_ref, b_ref, o_ref, slope=slope,
                 in_bias=ib_ref, in_slope=in_slope)

    return pl.pallas_call(
        body,
        grid=grid,
        in_specs=in_specs,
        out_specs=pl.BlockSpec((bm, bn), lambda i, j: (i, j)),
        out_shape=jax.ShapeDtypeStruct((m, n), jnp.float32),
    )(*args)


def _winv_body(p_ref, o_ref):
    e = p_ref[0] + p_ref[1]
    w = 1.0 / (e + 1e-16)
    col = lax.broadcasted_iota(jnp.int32, w.shape, 1)
    w = jnp.where(col < HEADS, w, 0.0)
    o_ref[...] = jnp.concatenate(
        [w, jnp.zeros((w.shape[0], RW - 8), jnp.float32)], axis=1)


def _winv(parts, bm=1000):
    return pl.pallas_call(
        _winv_body,
        grid=(NP // bm,),
        in_specs=[pl.BlockSpec((2, bm, 8), lambda i: (0, i, 0))],
        out_specs=pl.BlockSpec((bm, RW), lambda i: (i, 0)),
        out_shape=jax.ShapeDtypeStruct((NP, RW), jnp.float32),
    )(parts)


def _rowsum_body(g_ref, b_ref, o_ref):
    i = pl.program_id(0)
    x = g_ref[...] + b_ref[...]
    x = jnp.where(x >= 0, x, 0.01 * x)

    @pl.when(i == 0)
    def _():
        o_ref[...] = jnp.zeros_like(o_ref)

    o_ref[...] += jnp.sum(x, axis=0, keepdims=True)


def _rowsum(g, bias, bm=1000):
    m, n = g.shape
    return pl.pallas_call(
        _rowsum_body,
        grid=(m // bm,),
        in_specs=[
            pl.BlockSpec((bm, n), lambda i: (i, 0)),
            pl.BlockSpec((1, n), lambda i: (0, 0)),
        ],
        out_specs=pl.BlockSpec((1, n), lambda i: (0, 0)),
        out_shape=jax.ShapeDtypeStruct((1, n), jnp.float32),
    )(g, bias.reshape(1, n))


def _matvec_body(s_ref, w_ref, b_ref, o_ref):
    o_ref[...] = jnp.dot(s_ref[...] * (1.0 / N), w_ref[...],
                         preferred_element_type=jnp.float32) + b_ref[...]


def _matvec(s, w, b):
    k, n = w.shape
    return pl.pallas_call(
        _matvec_body,
        in_specs=[
            pl.BlockSpec((1, k), lambda: (0, 0)),
            pl.BlockSpec((k, n), lambda: (0, 0)),
            pl.BlockSpec((1, n), lambda: (0, 0)),
        ],
        out_specs=pl.BlockSpec((1, n), lambda: (0, 0)),
        out_shape=jax.ShapeDtypeStruct((1, n), jnp.float32),
    )(s, w, b.reshape(1, n))


def _sc_phase1(xl_lo, xl_hi, xr_lo, xr_hi, att2h, src_p, dst_p):
    mesh = plsc.VectorSubcoreMesh(core_axis_name="c", subcore_axis_name="s")

    @functools.partial(
        pl.kernel,
        out_type=(
            jax.ShapeDtypeStruct((EPAD, L), jnp.float32),
            jax.ShapeDtypeStruct((NC, 1024, RW), jnp.float32),
        ),
        mesh=mesh,
        scratch_types=[
            pltpu.VMEM((CH, FH), jnp.float32),
            pltpu.VMEM((CH, FH), jnp.float32),
            pltpu.VMEM((CH, FH), jnp.float32),
            pltpu.VMEM((CH, FH), jnp.float32),
            pltpu.VMEM((CH, FH), jnp.float32),
            pltpu.VMEM((CH, FH), jnp.float32),
            pltpu.VMEM((CH, FH), jnp.float32),
            pltpu.VMEM((CH, FH), jnp.float32),
            pltpu.VMEM((2 * FH,), jnp.float32),
            pltpu.VMEM((2 * CH,), jnp.int32),
            pltpu.VMEM((2 * CH,), jnp.int32),
            pltpu.VMEM((CH,), jnp.int32),
            pltpu.VMEM((CH,), jnp.int32),
            pltpu.VMEM((CH, RW), jnp.float32),
            pltpu.VMEM((CH, L), jnp.float32),
            pltpu.VMEM_SHARED((1024, RW), jnp.float32),
            pltpu.SemaphoreType.DMA,
            pltpu.SemaphoreType.DMA,
            pltpu.SemaphoreType.DMA,
        ],
    )
    def k(xlo_h, xhi_h, xrlo_h, xrhi_h, att_h, src_h, dst_h,
          ex_h, esum_h,
          alA, ahA, rlA, rhA, alB, ahB, rlB, rhB, attv,
          srcb, dstb, giA, giB, erow, exb, spm,
          semA, semB, semM):
        c = lax.axis_index("c")
        s = lax.axis_index("s")
        w = s * NC + c
        lanes = lax.iota(jnp.int32, L)
        zero16 = jnp.zeros((L,), jnp.float32)

        pltpu.sync_copy(att_h, attv)
        erow[...] = jnp.zeros_like(erow)

        for t in range(4):
            pltpu.sync_copy(erow, spm.at[pl.ds(s * 64 + t * CH, CH), :])
        plsc.subcore_barrier()

        def alpha_edge(i, base, xlo, xhi, xrlo, xrhi):
            parts = []
            for kk in range(HEADS):
                p = zero16
                for j in range(HALF // L):
                    sl = pl.ds(kk * HALF + j * L, L)
                    z = xlo[i, sl] + xrlo[i, sl]
                    p = p + jnp.maximum(z, 0.2 * z) * attv[sl]
                for j in range(HALF // L):
                    sl = pl.ds(kk * HALF + j * L, L)
                    z = xhi[i, sl] + xrhi[i, sl]
                    p = p + jnp.maximum(z, 0.2 * z) * attv[pl.ds(FH + kk * HALF + j * L, L)]
                parts.append(_lane_sum(p))
            a4 = zero16
            for kk in range(HEADS):
                a4 = a4 + jnp.where(lanes == kk, parts[kk], 0.0)
            exv = jnp.exp(a4)
            exv = jnp.where(lanes < HEADS, exv, 0.0)
            valid = (base + i) < E2
            exv = jnp.where(valid, exv, zero16)
            return exv

        def chunk(base, xlo, xhi, xrlo, xrhi, gi, dvec):
            dmod8 = (dvec & 15) * 8

            def edge_body(i, _):
                exv = alpha_edge(i, base, xlo, xhi, xrlo, xrhi)
                exb[i, pl.ds(0, L)] = exv
                dm = _bcast_lane(dmod8, i)
                exk = [_bcast_lane(exv, kk) for kk in range(HEADS)]
                for j in range(RW // L):
                    colj = lax.iota(jnp.int32, L) + j * L
                    row = jnp.zeros((L,), jnp.float32)
                    for kk in range(HEADS):
                        row = jnp.where(colj == dm + kk, exk[kk], row)
                    erow[i, pl.ds(j * L, L)] = row
                return 0
            lax.fori_loop(0, CH, edge_body, 0)
            pltpu.sync_copy(exb, ex_h.at[pl.ds(base, CH), :])
            pltpu.sync_copy(erow, spm.at[gi], add=True)

        def pair_body(u, _):
            base = pl.multiple_of(w * EW1 + u * (2 * CH), 2 * CH)
            pltpu.sync_copy(src_h.at[pl.ds(base, 2 * CH)], srcb)
            pltpu.sync_copy(dst_h.at[pl.ds(base, 2 * CH)], dstb)
            dA = dstb[pl.ds(0, CH)]
            dB = dstb[pl.ds(CH, CH)]
            giA[...] = lax.shift_right_logical(dA, 4)
            giB[...] = lax.shift_right_logical(dB, 4)
            cps = []
            for (buf, tbl, idx, sem) in (
                    (alA, xlo_h, srcb.at[pl.ds(0, CH)], semA),
                    (ahA, xhi_h, srcb.at[pl.ds(0, CH)], semA),
                    (rlA, xrlo_h, dstb.at[pl.ds(0, CH)], semA),
                    (rhA, xrhi_h, dstb.at[pl.ds(0, CH)], semA),
                    (alB, xlo_h, srcb.at[pl.ds(CH, CH)], semB),
                    (ahB, xhi_h, srcb.at[pl.ds(CH, CH)], semB),
                    (rlB, xrlo_h, dstb.at[pl.ds(CH, CH)], semB),
                    (rhB, xrhi_h, dstb.at[pl.ds(CH, CH)], semB)):
                cp = pltpu.make_async_copy(tbl.at[idx], buf, sem)
                cp.start()
                cps.append(cp)
            for cp in cps[:4]:
                cp.wait()
            chunk(base, alA, ahA, rlA, rhA, giA, dA)
            for cp in cps[4:]:
                cp.wait()
            chunk(base + CH, alB, ahB, rlB, rhB, giB, dB)
            return 0

        lax.fori_loop(0, EW1 // (2 * CH), pair_body, 0)
        plsc.subcore_barrier()
        pltpu.sync_copy(spm.at[pl.ds(s * 64, 64), :],
                        esum_h.at[c, pl.ds(s * 64, 64), :])

    return k(xl_lo, xl_hi, xr_lo, xr_hi, att2h, src_p, dst_p)


def _sc_phase2(xl_lo, xl_hi, winv, ex, src_p, dst_p):
    mesh = plsc.VectorSubcoreMesh(core_axis_name="c", subcore_axis_name="s")

    @functools.partial(
        pl.kernel,
        out_type=jax.ShapeDtypeStruct((NC, NP, RW), jnp.float32),
        mesh=mesh,
        scratch_types=[
            pltpu.VMEM((CH, FH), jnp.float32),
            pltpu.VMEM((CH, FH), jnp.float32),
            pltpu.VMEM((CH, RW), jnp.float32),
            pltpu.VMEM((CH, RW), jnp.float32),
            pltpu.VMEM((CH, L), jnp.float32),
            pltpu.VMEM((CH, L), jnp.float32),
            pltpu.VMEM((2 * CH,), jnp.int32),
            pltpu.VMEM((2 * CH,), jnp.int32),
            pltpu.VMEM((CH,), jnp.int32),
            pltpu.VMEM((CH,), jnp.int32),
            pltpu.VMEM((CH, RW), jnp.float32),
            pltpu.VMEM_SHARED((NP, RW), jnp.float32),
            pltpu.SemaphoreType.DMA,
            pltpu.SemaphoreType.DMA,
            pltpu.SemaphoreType.DMA,
        ],
    )
    def k(xlo_h, xhi_h, winv_h, ex_h, src_h, dst_h, g_h,
          xbA, xbB, wbA, wbB, ebA, ebB, srcb, dstb, giA, giB,
          rows, spm, semA, semB, semM):
        c = lax.axis_index("c")
        s = lax.axis_index("s")
        lanes = lax.iota(jnp.int32, L)

        rows[...] = jnp.zeros_like(rows)
        for t in range(NROW // CH):
            pltpu.sync_copy(rows, spm.at[pl.ds(s * NROW + t * CH, CH), :])
        plsc.subcore_barrier()

        def chunk(xb, wb, eb, gi):
            def edge_body(i, _):
                exv = eb[i, pl.ds(0, L)]
                wrow = wb[i, pl.ds(0, L)]
                wv = exv * wrow
                wk = [_bcast_lane(wv, kk) for kk in range(HEADS)]
                for j in range(HALF // L):
                    r = jnp.zeros((L,), jnp.float32)
                    for kk in range(HEADS):
                        r = r + wk[kk] * xb[i, pl.ds(kk * HALF + j * L, L)]
                    rows[i, pl.ds(j * L, L)] = r * 0.25
                return 0
            lax.fori_loop(0, CH, edge_body, 0)
            pltpu.sync_copy(rows, spm.at[gi], add=True)

        def make_pair_body(xl_h):
            def pair_body(u, _):
                base = pl.multiple_of(s * EW2 + u * (2 * CH), 2 * CH)
                pltpu.sync_copy(src_h.at[pl.ds(base, 2 * CH)], srcb)
                pltpu.sync_copy(dst_h.at[pl.ds(base, 2 * CH)], dstb)
                giA[...] = dstb[pl.ds(0, CH)]
                giB[...] = dstb[pl.ds(CH, CH)]
                cps = []
                for (buf, tbl, idx, sem) in (
                        (xbA, xl_h, srcb.at[pl.ds(0, CH)], semA),
                        (wbA, winv_h, giA, semA),
                        (xbB, xl_h, srcb.at[pl.ds(CH, CH)], semB),
                        (wbB, winv_h, giB, semB)):
                    cp = pltpu.make_async_copy(tbl.at[idx], buf, sem)
                    cp.start()
                    cps.append(cp)
                eA = pltpu.make_async_copy(
                    ex_h.at[pl.ds(base, CH), :], ebA, semA)
                eA.start()
                eB = pltpu.make_async_copy(
                    ex_h.at[pl.ds(base + CH, CH), :], ebB, semB)
                eB.start()
                cps[0].wait()
                cps[1].wait()
                eA.wait()
                chunk(xbA, wbA, ebA, giA)
                cps[2].wait()
                cps[3].wait()
                eB.wait()
                chunk(xbB, wbB, ebB, giB)
                return 0
            return pair_body

        npairs = EW2 // (2 * CH)

        @pl.when(c == 0)
        def _():
            lax.fori_loop(0, npairs, make_pair_body(xlo_h), 0)

        @pl.when(c == 1)
        def _():
            lax.fori_loop(0, npairs, make_pair_body(xhi_h), 0)

        plsc.subcore_barrier()
        pltpu.sync_copy(spm.at[pl.ds(s * NROW, NROW), :],
                        g_h.at[c, pl.ds(s * NROW, NROW), :])

    return k(xl_lo, xl_hi, winv, ex, src_p, dst_p)


def _half_cols(weight):
    wr = weight.reshape(weight.shape[:-1] + (HEADS, 2, HALF))
    lo = wr[..., :, 0, :].reshape(weight.shape[:-1] + (FH,))
    hi = wr[..., :, 1, :].reshape(weight.shape[:-1] + (FH,))
    return lo, hi


def _gat_layer(h, Wl, bl, Wr, br, att, src_p, dst_p, in_bias=None):
    Wl_lo, Wl_hi = _half_cols(Wl)
    bl_lo, bl_hi = _half_cols(bl)
    Wr_lo, Wr_hi = _half_cols(Wr)
    br_lo, br_hi = _half_cols(br)
    att_lo, att_hi = _half_cols(att.reshape(1, HEADS * HID))
    att2h = jnp.concatenate([att_lo.reshape(FH), att_hi.reshape(FH)])

    xl_lo = _matmul(h, Wl_lo, bl_lo, in_bias=in_bias)
    xl_hi = _matmul(h, Wl_hi, bl_hi, in_bias=in_bias)
    xr_lo = _matmul(h, Wr_lo, br_lo, in_bias=in_bias)
    xr_hi = _matmul(h, Wr_hi, br_hi, in_bias=in_bias)

    ex, esum_parts = _sc_phase1(xl_lo, xl_hi, xr_lo, xr_hi, att2h,
                                src_p, dst_p)
    esum_n = esum_parts.reshape(NC, 1024 * L, 8)[:, :NP, :]
    winv = _winv(esum_n)
    g_parts = _sc_phase2(xl_lo, xl_hi, winv, ex, src_p, dst_p)
    return jnp.concatenate([g_parts[0, :N], g_parts[1, :N]], axis=1)


def kernel(x, W0, b0, Wl1, bl1, Wr1, br1, att1, bias1, Wl2, bl2, Wr2, br2,
           att2, bias2, Wout, bout, edge_index):
    loop = jnp.arange(N, dtype=edge_index.dtype)
    src = jnp.concatenate([edge_index[0], loop])
    dst = jnp.concatenate([edge_index[1], loop])
    src_p = jnp.zeros((EPAD,), jnp.int32).at[:E2].set(src)
    dst_p = jnp.zeros((EPAD,), jnp.int32).at[:E2].set(dst)

    h = _matmul(x, W0, b0, slope=0.01)
    g1 = _gat_layer(h, Wl1, bl1, Wr1, br1, att1, src_p, dst_p)
    g2 = _gat_layer(g1, Wl2, bl2, Wr2, br2, att2, src_p, dst_p,
                    in_bias=bias1)
    s = _rowsum(g2, bias2)
    out = _matvec(s, Wout, bout)
    return out.reshape(OUT)


# speedup vs baseline: 7.6556x; 1.1169x over previous
import functools

import jax
import jax.numpy as jnp
from jax import lax
from jax.experimental import pallas as pl
from jax.experimental.pallas import tpu as pltpu
from jax.experimental.pallas import tpu_sc as plsc

N = 10000
IN = 256
HID = 256
OUT = 256
HEADS = 4
HALF = HID // 2
FH = HEADS * HALF

NC = 2
NS = 16
NW = NC * NS
L = 16
CH = 16
PAIR = 2 * CH
E2 = 170000
EPAD = 171008
EW1 = EPAD // NW
EW2 = EPAD // NS
NP = 10240
NROW = NP // NS
RW = 128
NPACK = 1024

_DN = lax.GatherDimensionNumbers(
    offset_dims=(), collapsed_slice_dims=(0,), start_index_map=(0,))


def _shuf(v, idx):
    return lax.gather(v, idx.reshape(L, 1), _DN, slice_sizes=(1,),
                      mode=lax.GatherScatterMode.PROMISE_IN_BOUNDS)


def _lane_sum(v):
    for sh in (8, 4, 2, 1):
        v = v + _shuf(v, lax.iota(jnp.int32, L) ^ sh)
    return v


def _bcast_lane(v, i):
    return _shuf(v, jnp.full((L,), i, jnp.int32))


def _mm_body(x_ref, w_ref, b_ref, o_ref, *, slope, in_bias, in_slope):
    x = x_ref[...]
    if in_bias is not None:
        x = x + in_bias[...]
        x = jnp.where(x >= 0, x, in_slope * x)
    acc = jnp.dot(x, w_ref[...], preferred_element_type=jnp.float32)
    acc = acc + b_ref[...]
    if slope is not None:
        acc = jnp.where(acc >= 0, acc, slope * acc)
    o_ref[...] = acc


def _matmul(x, w, b, slope=None, in_bias=None, in_slope=0.01,
            bm=1000, bn=512):
    m, k = x.shape
    n = w.shape[1]
    bn = min(bn, n)
    grid = (m // bm, n // bn)
    in_specs = [
        pl.BlockSpec((bm, k), lambda i, j: (i, 0)),
        pl.BlockSpec((k, bn), lambda i, j: (0, j)),
        pl.BlockSpec((1, bn), lambda i, j: (0, j)),
    ]
    args = [x, w, b.reshape(1, n)]
    if in_bias is not None:
        in_specs.append(pl.BlockSpec((1, k), lambda i, j: (0, 0)))
        args.append(in_bias.reshape(1, k))

    def body(x_ref, w_ref, b_ref, *rest):
        if in_bias is not None:
            ib_ref, o_ref = rest
        else:
            ib_ref, o_ref = None, rest[0]
        _mm_body(x_ref, w_ref, b_ref, o_ref, slope=slope,
                 in_bias=ib_ref, in_slope=in_slope)

    return pl.pallas_call(
        body,
        grid=grid,
        in_specs=in_specs,
        out_specs=pl.BlockSpec((bm, bn), lambda i, j: (i, j)),
        out_shape=jax.ShapeDtypeStruct((m, n), jnp.float32),
    )(*args)


def _winv_body(p_ref, o_ref):
    e = p_ref[0] + p_ref[1]
    w = 1.0 / (e + 1e-16)
    col = lax.broadcasted_iota(jnp.int32, w.shape, 1)
    w = jnp.where(col < HEADS, w, 0.0)
    o_ref[...] = jnp.concatenate(
        [w, jnp.zeros((w.shape[0], RW - 8), jnp.float32)], axis=1)


def _winv(parts, bm=1024):
    return pl.pallas_call(
        _winv_body,
        grid=(NP // bm,),
        in_specs=[pl.BlockSpec((2, bm, 8), lambda i: (0, i, 0))],
        out_specs=pl.BlockSpec((bm, RW), lambda i: (i, 0)),
        out_shape=jax.ShapeDtypeStruct((NP, RW), jnp.float32),
    )(parts)


def _rowsum_body(g_ref, b_ref, o_ref):
    i = pl.program_id(0)
    x = g_ref[...] + b_ref[...]
    x = jnp.where(x >= 0, x, 0.01 * x)

    @pl.when(i == 0)
    def _():
        o_ref[...] = jnp.zeros_like(o_ref)

    o_ref[...] += jnp.sum(x, axis=0, keepdims=True)


def _rowsum(g, bias, bm=1000):
    m, n = g.shape
    return pl.pallas_call(
        _rowsum_body,
        grid=(m // bm,),
        in_specs=[
            pl.BlockSpec((bm, n), lambda i: (i, 0)),
            pl.BlockSpec((1, n), lambda i: (0, 0)),
        ],
        out_specs=pl.BlockSpec((1, n), lambda i: (0, 0)),
        out_shape=jax.ShapeDtypeStruct((1, n), jnp.float32),
    )(g, bias.reshape(1, n))


def _matvec_body(s_ref, w_ref, b_ref, o_ref):
    o_ref[...] = jnp.dot(s_ref[...] * (1.0 / N), w_ref[...],
                         preferred_element_type=jnp.float32) + b_ref[...]


def _matvec(s, w, b):
    k, n = w.shape
    return pl.pallas_call(
        _matvec_body,
        in_specs=[
            pl.BlockSpec((1, k), lambda: (0, 0)),
            pl.BlockSpec((k, n), lambda: (0, 0)),
            pl.BlockSpec((1, n), lambda: (0, 0)),
        ],
        out_specs=pl.BlockSpec((1, n), lambda: (0, 0)),
        out_shape=jax.ShapeDtypeStruct((1, n), jnp.float32),
    )(s, w, b.reshape(1, n))


def _sc_phase1(xl_lo, xl_hi, xr_lo, xr_hi, att2h, sd_p):
    mesh = plsc.VectorSubcoreMesh(core_axis_name="c", subcore_axis_name="s")

    @functools.partial(
        pl.kernel,
        out_type=(
            jax.ShapeDtypeStruct((EPAD, L), jnp.float32),
            jax.ShapeDtypeStruct((NC, NPACK, RW), jnp.float32),
        ),
        mesh=mesh,
        scratch_types=[
            pltpu.VMEM((CH, FH), jnp.float32),
            pltpu.VMEM((CH, FH), jnp.float32),
            pltpu.VMEM((CH, FH), jnp.float32),
            pltpu.VMEM((CH, FH), jnp.float32),
            pltpu.VMEM((CH, FH), jnp.float32),
            pltpu.VMEM((CH, FH), jnp.float32),
            pltpu.VMEM((CH, FH), jnp.float32),
            pltpu.VMEM((CH, FH), jnp.float32),
            pltpu.VMEM((2 * FH,), jnp.float32),
            pltpu.VMEM((2 * PAIR,), jnp.int32),
            pltpu.VMEM((2 * PAIR,), jnp.int32),
            pltpu.VMEM((CH,), jnp.int32),
            pltpu.VMEM((CH,), jnp.int32),
            pltpu.VMEM((CH, RW), jnp.float32),
            pltpu.VMEM((CH, RW), jnp.float32),
            pltpu.VMEM((CH, L), jnp.float32),
            pltpu.VMEM((CH, L), jnp.float32),
            pltpu.VMEM_SHARED((NPACK, RW), jnp.float32),
            pltpu.SemaphoreType.DMA,
            pltpu.SemaphoreType.DMA,
            pltpu.SemaphoreType.DMA,
            pltpu.SemaphoreType.DMA,
            pltpu.SemaphoreType.DMA,
        ],
    )
    def k(xlo_h, xhi_h, xrlo_h, xrhi_h, att_h, sd_h,
          ex_h, esum_h,
          alA, ahA, rlA, rhA, alB, ahB, rlB, rhB, attv,
          sdb0, sdb1, giA, giB, erowA, erowB, exbA, exbB, spm,
          semA, semB, semI, semXA, semXB):
        c = lax.axis_index("c")
        s = lax.axis_index("s")
        w = s * NC + c
        e0 = w * EW1
        lanes = lax.iota(jnp.int32, L)
        zero16 = jnp.zeros((L,), jnp.float32)

        pltpu.sync_copy(att_h, attv)
        erowA[...] = jnp.zeros_like(erowA)

        for t in range(NPACK // NS // CH):
            pltpu.sync_copy(erowA, spm.at[pl.ds(s * (NPACK // NS) + t * CH, CH), :])
        plsc.subcore_barrier()

        def issue_gathers(base2):
            cps = []
            for (buf, tbl, off, sem) in (
                    (alA, xlo_h, 0, semA),
                    (ahA, xhi_h, 0, semA),
                    (rlA, xrlo_h, CH, semA),
                    (rhA, xrhi_h, CH, semA),
                    (alB, xlo_h, PAIR, semB),
                    (ahB, xhi_h, PAIR, semB),
                    (rlB, xrlo_h, PAIR + CH, semB),
                    (rhB, xrhi_h, PAIR + CH, semB)):
                cp = pltpu.make_async_copy(
                    tbl.at[sdb0.at[pl.ds(off, CH)]], buf, sem)
                cp.start()
                cps.append(cp)
            return cps

        def wait_gathers(bufs_sem):
            for (buf, tbl, off, sem) in bufs_sem:
                pltpu.make_async_copy(
                    tbl.at[sdb0.at[pl.ds(off, CH)]], buf, sem).wait()

        gsetA = ((alA, xlo_h, 0, semA), (ahA, xhi_h, 0, semA),
                 (rlA, xrlo_h, CH, semA), (rhA, xrhi_h, CH, semA))
        gsetB = ((alB, xlo_h, PAIR, semB), (ahB, xhi_h, PAIR, semB),
                 (rlB, xrlo_h, PAIR + CH, semB), (rhB, xrhi_h, PAIR + CH, semB))

        def alpha_edge(i, base, xlo, xhi, xrlo, xrhi):
            parts = []
            for kk in range(HEADS):
                p = zero16
                for j in range(HALF // L):
                    sl = pl.ds(kk * HALF + j * L, L)
                    z = xlo[i, sl] + xrlo[i, sl]
                    p = p + jnp.maximum(z, 0.2 * z) * attv[sl]
                for j in range(HALF // L):
                    sl = pl.ds(kk * HALF + j * L, L)
                    z = xhi[i, sl] + xrhi[i, sl]
                    p = p + jnp.maximum(z, 0.2 * z) * attv[pl.ds(FH + kk * HALF + j * L, L)]
                parts.append(_lane_sum(p))
            a4 = zero16
            for kk in range(HEADS):
                a4 = a4 + jnp.where(lanes == kk, parts[kk], 0.0)
            exv = jnp.exp(a4)
            exv = jnp.where(lanes < HEADS, exv, 0.0)
            valid = (base + i) < E2
            exv = jnp.where(valid, exv, zero16)
            return exv

        def chunk(u, base, sdoff, xlo, xhi, xrlo, xrhi, gi, erow, exb,
                  semX):
            dvec = sdb0[pl.ds(sdoff + CH, CH)]
            gi[...] = lax.shift_right_logical(dvec, 4)
            dmod8 = (dvec & 15) * 8

            @pl.when(u > 0)
            def _():
                pltpu.make_async_copy(
                    exb, ex_h.at[pl.ds(base, CH), :], semX).wait()

            def edge_body(i, _):
                exv = alpha_edge(i, base, xlo, xhi, xrlo, xrhi)
                exb[i, pl.ds(0, L)] = exv
                dm = _bcast_lane(dmod8, i)
                exk = [_bcast_lane(exv, kk) for kk in range(HEADS)]
                for j in range(RW // L):
                    colj = lax.iota(jnp.int32, L) + j * L
                    row = jnp.zeros((L,), jnp.float32)
                    for kk in range(HEADS):
                        row = jnp.where(colj == dm + kk, exk[kk], row)
                    erow[i, pl.ds(j * L, L)] = row
                return 0
            lax.fori_loop(0, CH, edge_body, 0)
            pltpu.make_async_copy(
                exb, ex_h.at[pl.ds(base, CH), :], semX).start()
            pltpu.sync_copy(erow, spm.at[gi], add=True)

        npairs = EW1 // PAIR

        pltpu.sync_copy(sd_h.at[pl.ds(e0 * 2, 2 * PAIR)], sdb0)
        issue_gathers(e0 * 2)

        def pair_body(u, _):
            base = pl.multiple_of(e0 + u * PAIR, PAIR)
            nxt = pl.multiple_of((base + PAIR) * 2, 2 * PAIR)
            cpI = pltpu.make_async_copy(
                sd_h.at[pl.ds(nxt, 2 * PAIR)], sdb1, semI)
            cpI.start()
            wait_gathers(gsetA)
            chunk(u, base, 0, alA, ahA, rlA, rhA, giA, erowA, exbA, semXA)
            wait_gathers(gsetB)
            chunk(u, base + CH, PAIR, alB, ahB, rlB, rhB, giB, erowB, exbB,
                  semXB)
            cpI.wait()
            for t in range(4):
                sdb0[pl.ds(t * L, L)] = sdb1[pl.ds(t * L, L)]

            @pl.when(u + 1 < npairs)
            def _():
                issue_gathers(nxt)
            return 0

        lax.fori_loop(0, npairs, pair_body, 0)
        pltpu.make_async_copy(
            exbA, ex_h.at[pl.ds(e0, CH), :], semXA).wait()
        pltpu.make_async_copy(
            exbB, ex_h.at[pl.ds(e0, CH), :], semXB).wait()
        plsc.subcore_barrier()
        pltpu.sync_copy(spm.at[pl.ds(s * (NPACK // NS), NPACK // NS), :],
                        esum_h.at[c, pl.ds(s * (NPACK // NS), NPACK // NS), :])

    return k(xl_lo, xl_hi, xr_lo, xr_hi, att2h, sd_p)


def _sc_phase2(xl_lo, xl_hi, winv, ex, sd_p):
    mesh = plsc.VectorSubcoreMesh(core_axis_name="c", subcore_axis_name="s")

    @functools.partial(
        pl.kernel,
        out_type=jax.ShapeDtypeStruct((NC, NP, RW), jnp.float32),
        mesh=mesh,
        scratch_types=[
            pltpu.VMEM((CH, FH), jnp.float32),
            pltpu.VMEM((CH, FH), jnp.float32),
            pltpu.VMEM((CH, RW), jnp.float32),
            pltpu.VMEM((CH, RW), jnp.float32),
            pltpu.VMEM((CH, L), jnp.float32),
            pltpu.VMEM((CH, L), jnp.float32),
            pltpu.VMEM((2 * PAIR,), jnp.int32),
            pltpu.VMEM((2 * PAIR,), jnp.int32),
            pltpu.VMEM((CH,), jnp.int32),
            pltpu.VMEM((CH,), jnp.int32),
            pltpu.VMEM((CH, RW), jnp.float32),
            pltpu.VMEM_SHARED((NP, RW), jnp.float32),
            pltpu.SemaphoreType.DMA,
            pltpu.SemaphoreType.DMA,
            pltpu.SemaphoreType.DMA,
        ],
    )
    def k(xlo_h, xhi_h, winv_h, ex_h, sd_h, g_h,
          xbA, xbB, wbA, wbB, ebA, ebB, sdb0, sdb1, giA, giB,
          rows, spm, semA, semB, semI):
        c = lax.axis_index("c")
        s = lax.axis_index("s")
        e0 = s * EW2

        rows[...] = jnp.zeros_like(rows)
        for t in range(NROW // CH):
            pltpu.sync_copy(rows, spm.at[pl.ds(s * NROW + t * CH, CH), :])
        plsc.subcore_barrier()

        def gsets(xl_h, base):
            setA = ((xbA, xl_h.at[sdb0.at[pl.ds(0, CH)]], semA),
                    (wbA, winv_h.at[sdb0.at[pl.ds(CH, CH)]], semA),
                    (ebA, ex_h.at[pl.ds(base, CH), :], semA))
            setB = ((xbB, xl_h.at[sdb0.at[pl.ds(PAIR, CH)]], semB),
                    (wbB, winv_h.at[sdb0.at[pl.ds(PAIR + CH, CH)]], semB),
                    (ebB, ex_h.at[pl.ds(base + CH, CH), :], semB))
            return setA, setB

        def issue(sets):
            for (buf, src, sem) in sets:
                pltpu.make_async_copy(src, buf, sem).start()

        def drain(sets):
            for (buf, src, sem) in sets:
                pltpu.make_async_copy(src, buf, sem).wait()

        def chunk(xb, wb, eb, gi, sdoff):
            gi[...] = sdb0[pl.ds(sdoff + CH, CH)]

            def edge_body(i, _):
                exv = eb[i, pl.ds(0, L)]
                wrow = wb[i, pl.ds(0, L)]
                wv = exv * wrow
                wk = [_bcast_lane(wv, kk) for kk in range(HEADS)]
                for j in range(HALF // L):
                    r = jnp.zeros((L,), jnp.float32)
                    for kk in range(HEADS):
                        r = r + wk[kk] * xb[i, pl.ds(kk * HALF + j * L, L)]
                    rows[i, pl.ds(j * L, L)] = r * 0.25
                return 0
            lax.fori_loop(0, CH, edge_body, 0)
            pltpu.sync_copy(rows, spm.at[gi], add=True)

        npairs = EW2 // PAIR

        def make_pair_body(xl_h):
            def pair_body(u, _):
                base = pl.multiple_of(e0 + u * PAIR, PAIR)
                nxt = pl.multiple_of((base + PAIR) * 2, 2 * PAIR)
                cpI = pltpu.make_async_copy(
                    sd_h.at[pl.ds(nxt, 2 * PAIR)], sdb1, semI)
                cpI.start()
                setA, setB = gsets(xl_h, base)
                drain(setA)
                chunk(xbA, wbA, ebA, giA, 0)
                drain(setB)
                chunk(xbB, wbB, ebB, giB, PAIR)
                cpI.wait()
                for t in range(4):
                    sdb0[pl.ds(t * L, L)] = sdb1[pl.ds(t * L, L)]

                @pl.when(u + 1 < npairs)
                def _():
                    nsetA, nsetB = gsets(xl_h, base + PAIR)
                    issue(nsetA)
                    issue(nsetB)
                return 0
            return pair_body

        def run(xl_h):
            pltpu.sync_copy(sd_h.at[pl.ds(e0 * 2, 2 * PAIR)], sdb0)
            setA, setB = gsets(xl_h, e0)
            issue(setA)
            issue(setB)
            lax.fori_loop(0, npairs, make_pair_body(xl_h), 0)

        @pl.when(c == 0)
        def _():
            run(xlo_h)

        @pl.when(c == 1)
        def _():
            run(xhi_h)

        plsc.subcore_barrier()
        pltpu.sync_copy(spm.at[pl.ds(s * NROW, NROW), :],
                        g_h.at[c, pl.ds(s * NROW, NROW), :])

    return k(xl_lo, xl_hi, winv, ex, sd_p)


def _half_cols(weight):
    wr = weight.reshape(weight.shape[:-1] + (HEADS, 2, HALF))
    lo = wr[..., :, 0, :].reshape(weight.shape[:-1] + (FH,))
    hi = wr[..., :, 1, :].reshape(weight.shape[:-1] + (FH,))
    return lo, hi


def _gat_layer(h, Wl, bl, Wr, br, att, sd_p, in_bias=None):
    Wl_lo, Wl_hi = _half_cols(Wl)
    bl_lo, bl_hi = _half_cols(bl)
    Wr_lo, Wr_hi = _half_cols(Wr)
    br_lo, br_hi = _half_cols(br)
    att_lo, att_hi = _half_cols(att.reshape(1, HEADS * HID))
    att2h = jnp.concatenate([att_lo.reshape(FH), att_hi.reshape(FH)])

    xl_lo = _matmul(h, Wl_lo, bl_lo, in_bias=in_bias)
    xl_hi = _matmul(h, Wl_hi, bl_hi, in_bias=in_bias)
    xr_lo = _matmul(h, Wr_lo, br_lo, in_bias=in_bias)
    xr_hi = _matmul(h, Wr_hi, br_hi, in_bias=in_bias)

    ex, esum_parts = _sc_phase1(xl_lo, xl_hi, xr_lo, xr_hi, att2h, sd_p)
    esum_n = esum_parts.reshape(NC, NPACK * L, 8)[:, :NP, :]
    winv = _winv(esum_n)
    g_parts = _sc_phase2(xl_lo, xl_hi, winv, ex, sd_p)
    return jnp.concatenate([g_parts[0, :N], g_parts[1, :N]], axis=1)


def kernel(x, W0, b0, Wl1, bl1, Wr1, br1, att1, bias1, Wl2, bl2, Wr2, br2,
           att2, bias2, Wout, bout, edge_index):
    loop = jnp.arange(N, dtype=edge_index.dtype)
    src = jnp.concatenate([edge_index[0], loop])
    dst = jnp.concatenate([edge_index[1], loop])
    src_p = jnp.zeros((EPAD + PAIR,), jnp.int32).at[:E2].set(src)
    dst_p = jnp.zeros((EPAD + PAIR,), jnp.int32).at[:E2].set(dst)
    sd_p = jnp.stack([src_p.reshape(-1, CH), dst_p.reshape(-1, CH)],
                     axis=1).reshape(-1)

    h = _matmul(x, W0, b0, slope=0.01)
    g1 = _gat_layer(h, Wl1, bl1, Wr1, br1, att1, sd_p)
    g2 = _gat_layer(g1, Wl2, bl2, Wr2, br2, att2, sd_p, in_bias=bias1)
    s = _rowsum(g2, bias2)
    out = _matvec(s, Wout, bout)
    return out.reshape(OUT)


# speedup vs baseline: 8.0127x; 1.0466x over previous
import functools

import jax
import jax.numpy as jnp
from jax import lax
from jax.experimental import pallas as pl
from jax.experimental.pallas import tpu as pltpu
from jax.experimental.pallas import tpu_sc as plsc

N = 10000
IN = 256
HID = 256
OUT = 256
HEADS = 4
HALF = HID // 2
FH = HEADS * HALF

NC = 2
NS = 16
NW = NC * NS
L = 16
CH = 16
PAIR = 2 * CH
E2 = 170000
EPAD = 171008
EW1 = EPAD // NW
EW2 = EPAD // NS
NP = 10240
NROW = NP // NS
RW = 128
NPACK = 1024

_DN = lax.GatherDimensionNumbers(
    offset_dims=(), collapsed_slice_dims=(0,), start_index_map=(0,))


def _shuf(v, idx):
    return lax.gather(v, idx.reshape(L, 1), _DN, slice_sizes=(1,),
                      mode=lax.GatherScatterMode.PROMISE_IN_BOUNDS)


def _lane_sum(v):
    for sh in (8, 4, 2, 1):
        v = v + _shuf(v, lax.iota(jnp.int32, L) ^ sh)
    return v


def _bcast_lane(v, i):
    return _shuf(v, jnp.full((L,), i, jnp.int32))


def _mm_body(x_ref, w_ref, b_ref, o_ref, *, slope, in_bias, in_slope):
    x = x_ref[...]
    if in_bias is not None:
        x = x + in_bias[...]
        x = jnp.where(x >= 0, x, in_slope * x)
    acc = jnp.dot(x, w_ref[...], preferred_element_type=jnp.float32)
    acc = acc + b_ref[...]
    if slope is not None:
        acc = jnp.where(acc >= 0, acc, slope * acc)
    o_ref[...] = acc


def _matmul(x, w, b, slope=None, in_bias=None, in_slope=0.01,
            bm=1000, bn=512):
    m, k = x.shape
    n = w.shape[1]
    bn = min(bn, n)
    grid = (m // bm, n // bn)
    in_specs = [
        pl.BlockSpec((bm, k), lambda i, j: (i, 0)),
        pl.BlockSpec((k, bn), lambda i, j: (0, j)),
        pl.BlockSpec((1, bn), lambda i, j: (0, j)),
    ]
    args = [x, w, b.reshape(1, n)]
    if in_bias is not None:
        in_specs.append(pl.BlockSpec((1, k), lambda i, j: (0, 0)))
        args.append(in_bias.reshape(1, k))

    def body(x_ref, w_ref, b_ref, *rest):
        if in_bias is not None:
            ib_ref, o_ref = rest
        else:
            ib_ref, o_ref = None, rest[0]
        _mm_body(x_ref, w_ref, b_ref, o_ref, slope=slope,
                 in_bias=ib_ref, in_slope=in_slope)

    return pl.pallas_call(
        body,
        grid=grid,
        in_specs=in_specs,
        out_specs=pl.BlockSpec((bm, bn), lambda i, j: (i, j)),
        out_shape=jax.ShapeDtypeStruct((m, n), jnp.float32),
    )(*args)


def _winv_body(p_ref, o_ref):
    e = p_ref[0] + p_ref[1]
    w = 1.0 / (e + 1e-16)
    col = lax.broadcasted_iota(jnp.int32, w.shape, 1)
    w = jnp.where(col < HEADS, w, 0.0)
    o_ref[...] = jnp.concatenate(
        [w, jnp.zeros((w.shape[0], RW - 8), jnp.float32)], axis=1)


def _winv(parts, bm=1024):
    return pl.pallas_call(
        _winv_body,
        grid=(NP // bm,),
        in_specs=[pl.BlockSpec((2, bm, 8), lambda i: (0, i, 0))],
        out_specs=pl.BlockSpec((bm, RW), lambda i: (i, 0)),
        out_shape=jax.ShapeDtypeStruct((NP, RW), jnp.float32),
    )(parts)


def _rowsum_body(g_ref, b_ref, o_ref):
    i = pl.program_id(0)
    x = g_ref[...] + b_ref[...]
    x = jnp.where(x >= 0, x, 0.01 * x)

    @pl.when(i == 0)
    def _():
        o_ref[...] = jnp.zeros_like(o_ref)

    o_ref[...] += jnp.sum(x, axis=0, keepdims=True)


def _rowsum(g, bias, bm=1000):
    m, n = g.shape
    return pl.pallas_call(
        _rowsum_body,
        grid=(m // bm,),
        in_specs=[
            pl.BlockSpec((bm, n), lambda i: (i, 0)),
            pl.BlockSpec((1, n), lambda i: (0, 0)),
        ],
        out_specs=pl.BlockSpec((1, n), lambda i: (0, 0)),
        out_shape=jax.ShapeDtypeStruct((1, n), jnp.float32),
    )(g, bias.reshape(1, n))


def _matvec_body(s_ref, w_ref, b_ref, o_ref):
    o_ref[...] = jnp.dot(s_ref[...] * (1.0 / N), w_ref[...],
                         preferred_element_type=jnp.float32) + b_ref[...]


def _matvec(s, w, b):
    k, n = w.shape
    return pl.pallas_call(
        _matvec_body,
        in_specs=[
            pl.BlockSpec((1, k), lambda: (0, 0)),
            pl.BlockSpec((k, n), lambda: (0, 0)),
            pl.BlockSpec((1, n), lambda: (0, 0)),
        ],
        out_specs=pl.BlockSpec((1, n), lambda: (0, 0)),
        out_shape=jax.ShapeDtypeStruct((1, n), jnp.float32),
    )(s, w, b.reshape(1, n))


def _sc_phase1(xl_lo, xl_hi, xr_lo, xr_hi, att2h, sd_p):
    mesh = plsc.VectorSubcoreMesh(core_axis_name="c", subcore_axis_name="s")

    @functools.partial(
        pl.kernel,
        out_type=(
            jax.ShapeDtypeStruct((EPAD, L), jnp.float32),
            jax.ShapeDtypeStruct((NC, NPACK, RW), jnp.float32),
        ),
        mesh=mesh,
        scratch_types=[
            pltpu.VMEM((CH, FH), jnp.float32),
            pltpu.VMEM((CH, FH), jnp.float32),
            pltpu.VMEM((CH, FH), jnp.float32),
            pltpu.VMEM((CH, FH), jnp.float32),
            pltpu.VMEM((CH, FH), jnp.float32),
            pltpu.VMEM((CH, FH), jnp.float32),
            pltpu.VMEM((CH, FH), jnp.float32),
            pltpu.VMEM((CH, FH), jnp.float32),
            pltpu.VMEM((2 * FH,), jnp.float32),
            pltpu.VMEM((2 * PAIR,), jnp.int32),
            pltpu.VMEM((2 * PAIR,), jnp.int32),
            pltpu.VMEM((CH,), jnp.int32),
            pltpu.VMEM((CH,), jnp.int32),
            pltpu.VMEM((CH, RW), jnp.float32),
            pltpu.VMEM((CH, RW), jnp.float32),
            pltpu.VMEM((CH, L), jnp.float32),
            pltpu.VMEM((CH, L), jnp.float32),
            pltpu.VMEM_SHARED((NPACK, RW), jnp.float32),
            pltpu.SemaphoreType.DMA,
            pltpu.SemaphoreType.DMA,
            pltpu.SemaphoreType.DMA,
            pltpu.SemaphoreType.DMA,
            pltpu.SemaphoreType.DMA,
            pltpu.SemaphoreType.DMA,
            pltpu.SemaphoreType.DMA,
        ],
    )
    def k(xlo_h, xhi_h, xrlo_h, xrhi_h, att_h, sd_h,
          ex_h, esum_h,
          alA, ahA, rlA, rhA, alB, ahB, rlB, rhB, attv,
          sdb0, sdb1, giA, giB, erowA, erowB, exbA, exbB, spm,
          semA, semB, semI, semXA, semXB, semSA, semSB):
        c = lax.axis_index("c")
        s = lax.axis_index("s")
        w = s * NC + c
        e0 = w * EW1
        lanes = lax.iota(jnp.int32, L)
        zero16 = jnp.zeros((L,), jnp.float32)

        pltpu.sync_copy(att_h, attv)
        erowA[...] = jnp.zeros_like(erowA)

        for t in range(NPACK // NS // CH):
            pltpu.sync_copy(erowA, spm.at[pl.ds(s * (NPACK // NS) + t * CH, CH), :])
        plsc.subcore_barrier()

        def issue_gathers(base2):
            cps = []
            for (buf, tbl, off, sem) in (
                    (alA, xlo_h, 0, semA),
                    (ahA, xhi_h, 0, semA),
                    (rlA, xrlo_h, CH, semA),
                    (rhA, xrhi_h, CH, semA),
                    (alB, xlo_h, PAIR, semB),
                    (ahB, xhi_h, PAIR, semB),
                    (rlB, xrlo_h, PAIR + CH, semB),
                    (rhB, xrhi_h, PAIR + CH, semB)):
                cp = pltpu.make_async_copy(
                    tbl.at[sdb0.at[pl.ds(off, CH)]], buf, sem)
                cp.start()
                cps.append(cp)
            return cps

        def wait_gathers(bufs_sem):
            for (buf, tbl, off, sem) in bufs_sem:
                pltpu.make_async_copy(
                    tbl.at[sdb0.at[pl.ds(off, CH)]], buf, sem).wait()

        gsetA = ((alA, xlo_h, 0, semA), (ahA, xhi_h, 0, semA),
                 (rlA, xrlo_h, CH, semA), (rhA, xrhi_h, CH, semA))
        gsetB = ((alB, xlo_h, PAIR, semB), (ahB, xhi_h, PAIR, semB),
                 (rlB, xrlo_h, PAIR + CH, semB), (rhB, xrhi_h, PAIR + CH, semB))

        def alpha_edge(i, base, xlo, xhi, xrlo, xrhi):
            parts = []
            for kk in range(HEADS):
                p = zero16
                for j in range(HALF // L):
                    sl = pl.ds(kk * HALF + j * L, L)
                    z = xlo[i, sl] + xrlo[i, sl]
                    p = p + jnp.maximum(z, 0.2 * z) * attv[sl]
                for j in range(HALF // L):
                    sl = pl.ds(kk * HALF + j * L, L)
                    z = xhi[i, sl] + xrhi[i, sl]
                    p = p + jnp.maximum(z, 0.2 * z) * attv[pl.ds(FH + kk * HALF + j * L, L)]
                parts.append(_lane_sum(p))
            a4 = zero16
            for kk in range(HEADS):
                a4 = a4 + jnp.where(lanes == kk, parts[kk], 0.0)
            exv = jnp.exp(a4)
            exv = jnp.where(lanes < HEADS, exv, 0.0)
            valid = (base + i) < E2
            exv = jnp.where(valid, exv, zero16)
            return exv

        def chunk(u, base, sdoff, xlo, xhi, xrlo, xrhi, gi, erow, exb,
                  semX, semS):
            dvec = sdb0[pl.ds(sdoff + CH, CH)]

            @pl.when(u > 0)
            def _():
                pltpu.make_async_copy(
                    exb, ex_h.at[pl.ds(base, CH), :], semX).wait()
                pltpu.make_async_copy(erow, spm.at[gi], semS).wait()

            gi[...] = lax.shift_right_logical(dvec, 4)
            dmod8 = (dvec & 15) * 8

            def edge_body(i, _):
                exv = alpha_edge(i, base, xlo, xhi, xrlo, xrhi)
                exb[i, pl.ds(0, L)] = exv
                dm = _bcast_lane(dmod8, i)
                exk = [_bcast_lane(exv, kk) for kk in range(HEADS)]
                for j in range(RW // L):
                    colj = lax.iota(jnp.int32, L) + j * L
                    row = jnp.zeros((L,), jnp.float32)
                    for kk in range(HEADS):
                        row = jnp.where(colj == dm + kk, exk[kk], row)
                    erow[i, pl.ds(j * L, L)] = row
                return 0
            lax.fori_loop(0, CH, edge_body, 0)
            pltpu.make_async_copy(
                exb, ex_h.at[pl.ds(base, CH), :], semX).start()
            pltpu.make_async_copy(erow, spm.at[gi], semS).start(add=True)

        npairs = EW1 // PAIR

        pltpu.sync_copy(sd_h.at[pl.ds(e0 * 2, 2 * PAIR)], sdb0)
        issue_gathers(e0 * 2)

        def pair_body(u, _):
            base = pl.multiple_of(e0 + u * PAIR, PAIR)
            nxt = pl.multiple_of((base + PAIR) * 2, 2 * PAIR)
            cpI = pltpu.make_async_copy(
                sd_h.at[pl.ds(nxt, 2 * PAIR)], sdb1, semI)
            cpI.start()
            wait_gathers(gsetA)
            chunk(u, base, 0, alA, ahA, rlA, rhA, giA, erowA, exbA,
                  semXA, semSA)
            wait_gathers(gsetB)
            chunk(u, base + CH, PAIR, alB, ahB, rlB, rhB, giB, erowB, exbB,
                  semXB, semSB)
            cpI.wait()
            for t in range(4):
                sdb0[pl.ds(t * L, L)] = sdb1[pl.ds(t * L, L)]

            @pl.when(u + 1 < npairs)
            def _():
                issue_gathers(nxt)
            return 0

        lax.fori_loop(0, npairs, pair_body, 0)
        pltpu.make_async_copy(
            exbA, ex_h.at[pl.ds(e0, CH), :], semXA).wait()
        pltpu.make_async_copy(
            exbB, ex_h.at[pl.ds(e0, CH), :], semXB).wait()
        pltpu.make_async_copy(erowA, spm.at[giA], semSA).wait()
        pltpu.make_async_copy(erowB, spm.at[giB], semSB).wait()
        plsc.subcore_barrier()
        pltpu.sync_copy(spm.at[pl.ds(s * (NPACK // NS), NPACK // NS), :],
                        esum_h.at[c, pl.ds(s * (NPACK // NS), NPACK // NS), :])

    return k(xl_lo, xl_hi, xr_lo, xr_hi, att2h, sd_p)


def _sc_phase2(xl_lo, xl_hi, winv, ex, sd_p):
    mesh = plsc.VectorSubcoreMesh(core_axis_name="c", subcore_axis_name="s")

    @functools.partial(
        pl.kernel,
        out_type=jax.ShapeDtypeStruct((NC, NP, RW), jnp.float32),
        mesh=mesh,
        scratch_types=[
            pltpu.VMEM((CH, FH), jnp.float32),
            pltpu.VMEM((CH, FH), jnp.float32),
            pltpu.VMEM((CH, RW), jnp.float32),
            pltpu.VMEM((CH, RW), jnp.float32),
            pltpu.VMEM((CH, L), jnp.float32),
            pltpu.VMEM((CH, L), jnp.float32),
            pltpu.VMEM((2 * PAIR,), jnp.int32),
            pltpu.VMEM((2 * PAIR,), jnp.int32),
            pltpu.VMEM((CH,), jnp.int32),
            pltpu.VMEM((CH,), jnp.int32),
            pltpu.VMEM((CH, RW), jnp.float32),
            pltpu.VMEM((CH, RW), jnp.float32),
            pltpu.VMEM_SHARED((NP, RW), jnp.float32),
            pltpu.SemaphoreType.DMA,
            pltpu.SemaphoreType.DMA,
            pltpu.SemaphoreType.DMA,
            pltpu.SemaphoreType.DMA,
            pltpu.SemaphoreType.DMA,
        ],
    )
    def k(xlo_h, xhi_h, winv_h, ex_h, sd_h, g_h,
          xbA, xbB, wbA, wbB, ebA, ebB, sdb0, sdb1, giA, giB,
          rowsA, rowsB, spm, semA, semB, semI, semSA, semSB):
        c = lax.axis_index("c")
        s = lax.axis_index("s")
        e0 = s * EW2

        rowsA[...] = jnp.zeros_like(rowsA)
        for t in range(NROW // CH):
            pltpu.sync_copy(rowsA, spm.at[pl.ds(s * NROW + t * CH, CH), :])
        plsc.subcore_barrier()

        def gsets(xl_h, base):
            setA = ((xbA, xl_h.at[sdb0.at[pl.ds(0, CH)]], semA),
                    (wbA, winv_h.at[sdb0.at[pl.ds(CH, CH)]], semA),
                    (ebA, ex_h.at[pl.ds(base, CH), :], semA))
            setB = ((xbB, xl_h.at[sdb0.at[pl.ds(PAIR, CH)]], semB),
                    (wbB, winv_h.at[sdb0.at[pl.ds(PAIR + CH, CH)]], semB),
                    (ebB, ex_h.at[pl.ds(base + CH, CH), :], semB))
            return setA, setB

        def issue(sets):
            for (buf, src, sem) in sets:
                pltpu.make_async_copy(src, buf, sem).start()

        def drain(sets):
            for (buf, src, sem) in sets:
                pltpu.make_async_copy(src, buf, sem).wait()

        def chunk(u, xb, wb, eb, gi, rows, semS, sdoff):
            @pl.when(u > 0)
            def _():
                pltpu.make_async_copy(rows, spm.at[gi], semS).wait()

            gi[...] = sdb0[pl.ds(sdoff + CH, CH)]

            def edge_body(i, _):
                exv = eb[i, pl.ds(0, L)]
                wrow = wb[i, pl.ds(0, L)]
                wv = exv * wrow
                wk = [_bcast_lane(wv, kk) for kk in range(HEADS)]
                for j in range(HALF // L):
                    r = jnp.zeros((L,), jnp.float32)
                    for kk in range(HEADS):
                        r = r + wk[kk] * xb[i, pl.ds(kk * HALF + j * L, L)]
                    rows[i, pl.ds(j * L, L)] = r * 0.25
                return 0
            lax.fori_loop(0, CH, edge_body, 0)
            pltpu.make_async_copy(rows, spm.at[gi], semS).start(add=True)

        npairs = EW2 // PAIR

        def make_pair_body(xl_h):
            def pair_body(u, _):
                base = pl.multiple_of(e0 + u * PAIR, PAIR)
                nxt = pl.multiple_of((base + PAIR) * 2, 2 * PAIR)
                cpI = pltpu.make_async_copy(
                    sd_h.at[pl.ds(nxt, 2 * PAIR)], sdb1, semI)
                cpI.start()
                setA, setB = gsets(xl_h, base)
                drain(setA)
                chunk(u, xbA, wbA, ebA, giA, rowsA, semSA, 0)
                drain(setB)
                chunk(u, xbB, wbB, ebB, giB, rowsB, semSB, PAIR)
                cpI.wait()
                for t in range(4):
                    sdb0[pl.ds(t * L, L)] = sdb1[pl.ds(t * L, L)]

                @pl.when(u + 1 < npairs)
                def _():
                    nsetA, nsetB = gsets(xl_h, base + PAIR)
                    issue(nsetA)
                    issue(nsetB)
                return 0
            return pair_body

        def run(xl_h):
            pltpu.sync_copy(sd_h.at[pl.ds(e0 * 2, 2 * PAIR)], sdb0)
            setA, setB = gsets(xl_h, e0)
            issue(setA)
            issue(setB)
            lax.fori_loop(0, npairs, make_pair_body(xl_h), 0)
            pltpu.make_async_copy(rowsA, spm.at[giA], semSA).wait()
            pltpu.make_async_copy(rowsB, spm.at[giB], semSB).wait()

        @pl.when(c == 0)
        def _():
            run(xlo_h)

        @pl.when(c == 1)
        def _():
            run(xhi_h)

        plsc.subcore_barrier()
        pltpu.sync_copy(spm.at[pl.ds(s * NROW, NROW), :],
                        g_h.at[c, pl.ds(s * NROW, NROW), :])

    return k(xl_lo, xl_hi, winv, ex, sd_p)


def _half_cols(weight):
    wr = weight.reshape(weight.shape[:-1] + (HEADS, 2, HALF))
    lo = wr[..., :, 0, :].reshape(weight.shape[:-1] + (FH,))
    hi = wr[..., :, 1, :].reshape(weight.shape[:-1] + (FH,))
    return lo, hi


def _gat_layer(h, Wl, bl, Wr, br, att, sd_p, in_bias=None):
    Wl_lo, Wl_hi = _half_cols(Wl)
    bl_lo, bl_hi = _half_cols(bl)
    Wr_lo, Wr_hi = _half_cols(Wr)
    br_lo, br_hi = _half_cols(br)
    att_lo, att_hi = _half_cols(att.reshape(1, HEADS * HID))
    att2h = jnp.concatenate([att_lo.reshape(FH), att_hi.reshape(FH)])

    xl_lo = _matmul(h, Wl_lo, bl_lo, in_bias=in_bias)
    xl_hi = _matmul(h, Wl_hi, bl_hi, in_bias=in_bias)
    xr_lo = _matmul(h, Wr_lo, br_lo, in_bias=in_bias)
    xr_hi = _matmul(h, Wr_hi, br_hi, in_bias=in_bias)

    ex, esum_parts = _sc_phase1(xl_lo, xl_hi, xr_lo, xr_hi, att2h, sd_p)
    esum_n = esum_parts.reshape(NC, NPACK * L, 8)[:, :NP, :]
    winv = _winv(esum_n)
    g_parts = _sc_phase2(xl_lo, xl_hi, winv, ex, sd_p)
    return jnp.concatenate([g_parts[0, :N], g_parts[1, :N]], axis=1)


def kernel(x, W0, b0, Wl1, bl1, Wr1, br1, att1, bias1, Wl2, bl2, Wr2, br2,
           att2, bias2, Wout, bout, edge_index):
    loop = jnp.arange(N, dtype=edge_index.dtype)
    src = jnp.concatenate([edge_index[0], loop])
    dst = jnp.concatenate([edge_index[1], loop])
    src_p = jnp.zeros((EPAD + PAIR,), jnp.int32).at[:E2].set(src)
    dst_p = jnp.zeros((EPAD + PAIR,), jnp.int32).at[:E2].set(dst)
    sd_p = jnp.stack([src_p.reshape(-1, CH), dst_p.reshape(-1, CH)],
                     axis=1).reshape(-1)

    h = _matmul(x, W0, b0, slope=0.01)
    g1 = _gat_layer(h, Wl1, bl1, Wr1, br1, att1, sd_p)
    g2 = _gat_layer(g1, Wl2, bl2, Wr2, br2, att2, sd_p, in_bias=bias1)
    s = _rowsum(g2, bias2)
    out = _matvec(s, Wout, bout)
    return out.reshape(OUT)
